# Initial kernel scaffold; baseline (speedup 1.0000x reference)
import jax
import jax.numpy as jnp
from jax.experimental import pallas as pl


def kernel(x, node_time, edge_time, edge_weight, dense_features, W_self1, W_msg1, b1, W_self2, W_msg2, b2, emb_table, cross_w, cross_b, W_mlp1, b_mlp1, W_mlp2, b_mlp2, W_out, b_out, node_id, edge_index, sparse_features):
    raise NotImplementedError("write your pallas kernel here")


# speedup vs baseline: 3.3935x; 3.3935x over previous
import functools

import jax
import jax.numpy as jnp
from jax import lax
from jax.experimental import pallas as pl
from jax.experimental.pallas import tpu as pltpu
from jax.experimental.pallas import tpu_sc as plsc

_NC = 2
_NS = 16
_NW = _NC * _NS
_LANES = 16


def _ceil_to(a, m):
    return (a + m - 1) // m * m


def _sc_edge_call(xtab, src, dst, et, ew, ids, ts, w_in, compute_w):
    n, d = xtab.shape
    e_pad = src.shape[0]
    per_tile = e_pad // _NW
    chunk = 128
    nchunks = per_tile // chunk
    rows_per_sub = (n // (8 * _NS)) * 8
    tail = n - rows_per_sub * _NS
    zrows = 48
    n_flat = ids.shape[0] if compute_w else 0

    mesh = plsc.VectorSubcoreMesh(core_axis_name="c", subcore_axis_name="s")

    out_type = [jax.ShapeDtypeStruct((_NC, n, d), jnp.float32)]
    if compute_w:
        out_type.append(jax.ShapeDtypeStruct((e_pad,), jnp.float32))

    scratch = [
        pltpu.VMEM_SHARED((n, d), jnp.float32),
        pltpu.VMEM((zrows, d), jnp.float32),
        pltpu.VMEM((1, chunk), jnp.int32),
        pltpu.VMEM((1, chunk), jnp.int32),
        pltpu.VMEM((1, chunk), jnp.float32),
        pltpu.VMEM((1, chunk, d), jnp.float32),
        pltpu.VMEM((per_tile,), jnp.float32),
        pltpu.SemaphoreType.DMA,
    ]
    if compute_w:
        scratch += [
            pltpu.VMEM((1, chunk), jnp.float32),
            pltpu.VMEM((1, chunk), jnp.float32),
            pltpu.VMEM((n,), jnp.float32),
            pltpu.VMEM((1, chunk), jnp.int32),
            pltpu.VMEM((1, chunk), jnp.float32),
        ]

    def body(*refs):
        zero16 = jnp.zeros((_LANES,), jnp.float32)
        if compute_w:
            (x_hbm, src_hbm, dst_hbm, et_hbm, ew_hbm, ids_hbm, ts_hbm,
             agg_hbm, w_hbm,
             acc, zbuf, srcb, dstb, wb, rows, wall, sem,
             etb, ewb, node_t, idb, tsb) = refs
        else:
            (x_hbm, src_hbm, dst_hbm, w_in_hbm,
             agg_hbm,
             acc, zbuf, srcb, dstb, wb, rows, wall, sem) = refs

        c = lax.axis_index("c")
        s = lax.axis_index("s")
        wid = c * _NS + s

        def zb(i, _):
            for q in range(d // _LANES):
                zbuf[i, pl.ds(q * _LANES, _LANES)] = zero16
            return 0
        lax.fori_loop(0, zrows, zb, 0)
        row0 = s * rows_per_sub
        for k in range(rows_per_sub // zrows):
            pltpu.sync_copy(zbuf, acc.at[pl.ds(row0 + k * zrows, zrows)])

        @pl.when(s == _NS - 1)
        def _():
            pltpu.sync_copy(zbuf.at[pl.ds(0, tail)],
                            acc.at[pl.ds(rows_per_sub * _NS, tail)])
        plsc.subcore_barrier()

        if compute_w:
            def zt(i, _):
                node_t[pl.ds(i * _LANES, _LANES)] = zero16
                return 0
            lax.fori_loop(0, n // _LANES, zt, 0)

            def smax(i, _):
                pltpu.sync_copy(ids_hbm.at[pl.ds(i * chunk, chunk)],
                                idb.at[0])
                pltpu.sync_copy(ts_hbm.at[pl.ds(i * chunk, chunk)],
                                tsb.at[0])
                for g in range(chunk // _LANES):
                    sl = pl.ds(g * _LANES, _LANES)
                    ids16 = idb[0, sl]
                    ts16 = tsb[0, sl]

                    def rmw(p, ids16=ids16, ts16=ts16):
                        cur = plsc.load_gather(node_t, [ids16], mask=p)
                        new = jnp.maximum(cur, ts16)
                        plsc.store_scatter(node_t, [ids16], new, mask=p)
                        got = plsc.load_gather(node_t, [ids16], mask=p)
                        return jnp.logical_and(p, got < new)

                    pending = rmw(ids16 >= 0)

                    @pl.when(jnp.sum(pending.astype(jnp.int32)) > 0)
                    def _(pending=pending, rmw=rmw):
                        lax.fori_loop(0, _LANES - 1,
                                      lambda j, p: rmw(p), pending)
                return 0
            lax.fori_loop(0, n_flat // chunk, smax, 0)

        base_t = wid * per_tile

        def chunk_body(i, _):
            base = base_t + i * chunk
            pltpu.sync_copy(src_hbm.at[pl.ds(base, chunk)], srcb.at[0])
            pltpu.sync_copy(dst_hbm.at[pl.ds(base, chunk)], dstb.at[0])
            if compute_w:
                pltpu.sync_copy(et_hbm.at[pl.ds(base, chunk)], etb.at[0])
                pltpu.sync_copy(ew_hbm.at[pl.ds(base, chunk)], ewb.at[0])
                for q in range(chunk // _LANES):
                    sl = pl.ds(q * _LANES, _LANES)
                    d16 = dstb[0, sl]
                    nt = plsc.load_gather(node_t, [d16])
                    w16 = ewb[0, sl] * jnp.exp(-jnp.abs(nt - etb[0, sl]))
                    wb[0, sl] = w16
                    wall[pl.ds(i * chunk + q * _LANES, _LANES)] = w16
            else:
                pltpu.sync_copy(w_in_hbm.at[pl.ds(base, chunk)], wb.at[0])

            pltpu.async_copy(x_hbm.at[srcb.at[0]], rows.at[0], sem).wait()

            def egrp(g, _):
                w16 = wb[0, pl.ds(g * _LANES, _LANES)]
                for j in range(_LANES):
                    w_s = w16[j]
                    e = g * _LANES + j
                    for q in range(d // _LANES):
                        sl = pl.ds(q * _LANES, _LANES)
                        rows[0, e, sl] = rows[0, e, sl] * w_s
                return 0
            lax.fori_loop(0, chunk // _LANES, egrp, 0)

            pltpu.sync_copy(rows.at[0], acc.at[dstb.at[0]], add=True)
            return 0
        lax.fori_loop(0, nchunks, chunk_body, 0)

        if compute_w:
            pltpu.sync_copy(wall, w_hbm.at[pl.ds(base_t, per_tile)])

        plsc.subcore_barrier()

        for k in range(rows_per_sub // zrows):
            r0 = row0 + k * zrows
            pltpu.sync_copy(acc.at[pl.ds(r0, zrows)], zbuf)
            pltpu.sync_copy(zbuf, agg_hbm.at[c, pl.ds(r0, zrows)])

        @pl.when(s == _NS - 1)
        def _():
            r0 = rows_per_sub * _NS
            pltpu.sync_copy(acc.at[pl.ds(r0, tail)], zbuf.at[pl.ds(0, tail)])
            pltpu.sync_copy(zbuf.at[pl.ds(0, tail)],
                            agg_hbm.at[c, pl.ds(r0, tail)])

    call = pl.kernel(body, out_type=tuple(out_type), mesh=mesh,
                     compiler_params=pltpu.CompilerParams(
                         needs_layout_passes=False),
                     scratch_types=scratch)
    if compute_w:
        return call(xtab, src, dst, et, ew, ids, ts)
    return call(xtab, src, dst, w_in)


def _sc_gather_call(h2, flat_ids, emb_wide, sf_flat, off_flat, de):
    n, d = h2.shape
    n_t = flat_ids.shape[0]
    n_e = sf_flat.shape[0]
    per_row = 128 // de
    chunk = 128
    chunks_t = n_t // chunk
    chunks_e = n_e // chunk

    mesh = plsc.VectorSubcoreMesh(core_axis_name="c", subcore_axis_name="s")
    out_type = (
        jax.ShapeDtypeStruct((n_t, d), jnp.float32),
        jax.ShapeDtypeStruct((n_e, de), jnp.float32),
    )
    scratch = [
        pltpu.VMEM((1, chunk), jnp.int32),
        pltpu.VMEM((chunk, d), jnp.float32),
        pltpu.VMEM((1, chunk), jnp.int32),
        pltpu.VMEM((1, chunk), jnp.int32),
        pltpu.VMEM((1, chunk), jnp.int32),
        pltpu.VMEM((1, chunk), jnp.int32),
        pltpu.VMEM((chunk, 128), jnp.float32),
        pltpu.VMEM((chunk, de), jnp.float32),
        pltpu.SemaphoreType.DMA,
    ]

    def body(h2_hbm, ids_hbm, emb_hbm, sf_hbm, off_hbm,
             temb_hbm, embr_hbm,
             idxb, rowst, sfb, ofb, eidx, erem, rowse, outbuf, sem):
        c = lax.axis_index("c")
        s = lax.axis_index("s")
        wid = c * _NS + s

        nrep = max(-(-chunks_t // _NW), -(-chunks_e // _NW))
        for rep in range(nrep):
            cid = wid + _NW * rep

            @pl.when(cid < chunks_t)
            def _(cid=cid):
                pltpu.sync_copy(ids_hbm.at[pl.ds(cid * chunk, chunk)],
                                idxb.at[0])
                pltpu.async_copy(h2_hbm.at[idxb.at[0]], rowst, sem).wait()
                pltpu.sync_copy(rowst, temb_hbm.at[pl.ds(cid * chunk, chunk)])

            @pl.when(cid < chunks_e)
            def _(cid=cid):
                pltpu.sync_copy(sf_hbm.at[pl.ds(cid * chunk, chunk)],
                                sfb.at[0])
                pltpu.sync_copy(off_hbm.at[pl.ds(cid * chunk, chunk)],
                                ofb.at[0])
                for q in range(chunk // _LANES):
                    sl = pl.ds(q * _LANES, _LANES)
                    v16 = sfb[0, sl] + ofb[0, sl]
                    eidx[0, sl] = v16 // per_row
                    erem[0, sl] = (v16 % per_row) * de
                pltpu.async_copy(emb_hbm.at[eidx.at[0]], rowse, sem).wait()
                for g in range(chunk // _LANES):
                    rem16 = erem[0, pl.ds(g * _LANES, _LANES)]
                    for j in range(_LANES):
                        k = g * _LANES + j
                        outbuf[k, :] = rowse[k, pl.ds(rem16[j], de)]
                pltpu.sync_copy(outbuf,
                                embr_hbm.at[pl.ds(cid * chunk, chunk)])

    call = pl.kernel(body, out_type=out_type, mesh=mesh,
                     compiler_params=pltpu.CompilerParams(
                         needs_layout_passes=False),
                     scratch_types=scratch)
    return call(h2, flat_ids, emb_wide, sf_flat, off_flat)


def _tc_affine(xin, a0, a1, ws, wm, b):
    n, d = xin.shape
    r = 1000

    def body(x_ref, a0_ref, a1_ref, ws_ref, wm_ref, b_ref, o_ref):
        agg = a0_ref[...] + a1_ref[...]
        acc = jnp.dot(x_ref[...], ws_ref[...],
                      preferred_element_type=jnp.float32)
        acc = acc + jnp.dot(agg, wm_ref[...],
                            preferred_element_type=jnp.float32)
        o_ref[...] = jnp.maximum(acc + b_ref[...], 0.0)

    return pl.pallas_call(
        body,
        grid=(n // r,),
        in_specs=[
            pl.BlockSpec((r, d), lambda i: (i, 0)),
            pl.BlockSpec((r, d), lambda i: (i, 0)),
            pl.BlockSpec((r, d), lambda i: (i, 0)),
            pl.BlockSpec((d, d), lambda i: (0, 0)),
            pl.BlockSpec((d, d), lambda i: (0, 0)),
            pl.BlockSpec((1, d), lambda i: (0, 0)),
        ],
        out_specs=pl.BlockSpec((r, d), lambda i: (i, 0)),
        out_shape=jax.ShapeDtypeStruct((n, d), jnp.float32),
    )(xin, a0, a1, ws, wm, b.reshape(1, d))


def _tc_head(x0, cross_w, cross_b, w1, b1, w2, b2, wo, bo):
    bsz, dc = x0.shape
    dh1 = w1.shape[1]
    dh2 = w2.shape[1]
    do = wo.shape[1]
    n_cross = cross_w.shape[0]

    def body(x0_ref, cw_ref, cb_ref, w1_ref, b1_ref, w2_ref, b2_ref,
             wo_ref, bo_ref, o_ref):
        x0v = x0_ref[...]
        cw = cw_ref[...]
        cb = cb_ref[...]
        xl = x0v
        for i in range(n_cross):
            cwq = cw[i].astype(jnp.bfloat16).astype(jnp.float32)
            s = jnp.dot(xl, cwq.reshape(-1, 1),
                        preferred_element_type=jnp.float32)
            xl = x0v * s + cb[i][None, :] + xl
        hm = jnp.maximum(jnp.dot(xl, w1_ref[...],
                                 preferred_element_type=jnp.float32)
                         + b1_ref[...], 0.0)
        hm = hm.astype(jnp.bfloat16).astype(jnp.float32)
        hm = jnp.maximum(jnp.dot(hm, w2_ref[...],
                                 preferred_element_type=jnp.float32)
                         + b2_ref[...], 0.0)
        hm = hm.astype(jnp.bfloat16).astype(jnp.float32)
        logit = jnp.dot(hm, wo_ref[...],
                        preferred_element_type=jnp.float32) + bo_ref[...]
        o_ref[...] = jax.nn.sigmoid(logit)

    return pl.pallas_call(
        body,
        out_shape=jax.ShapeDtypeStruct((bsz, do), jnp.float32),
    )(x0, cross_w, cross_b, w1, b1.reshape(1, dh1), w2, b2.reshape(1, dh2),
      wo, bo.reshape(1, do))


def kernel(x, node_time, edge_time, edge_weight, dense_features,
           W_self1, W_msg1, b1, W_self2, W_msg2, b2,
           emb_table, cross_w, cross_b, W_mlp1, b_mlp1, W_mlp2, b_mlp2,
           W_out, b_out, node_id, edge_index, sparse_features):
    n, d = x.shape
    bsz, hist = node_time.shape
    n_edges = edge_time.shape[0]
    n_fields = sparse_features.shape[1]
    vocab = emb_table.shape[0] // n_fields

    dst0 = edge_index[1].astype(jnp.int32)
    perm = jnp.argsort(dst0, stable=True)
    src_s = edge_index[0].astype(jnp.int32)[perm]
    dst_s = dst0[perm]
    et_s = edge_time[perm]
    ew_s = edge_weight[perm]

    e_pad = _ceil_to(n_edges, _NW * 128)
    pad = e_pad - n_edges
    src = jnp.pad(src_s, (0, pad))
    dst = jnp.pad(dst_s, (0, pad))
    et = jnp.pad(et_s, (0, pad))
    ew = jnp.pad(ew_s, (0, pad))

    flat_ids = node_id.reshape(-1).astype(jnp.int32)
    flat_times = node_time.reshape(-1)

    aggp, w_e = _sc_edge_call(x, src, dst, et, ew, flat_ids, flat_times,
                              None, True)
    h = _tc_affine(x, aggp[0], aggp[1], W_self1, W_msg1, b1)

    (agg2p,) = _sc_edge_call(h, src, dst, None, None, None, None, w_e, False)
    h2 = _tc_affine(h, agg2p[0], agg2p[1], W_self2, W_msg2, b2)

    offs = (jnp.zeros((bsz, n_fields), jnp.int32)
            + jnp.arange(n_fields, dtype=jnp.int32)[None, :] * vocab)
    sf_flat = sparse_features.astype(jnp.int32).reshape(-1)
    off_flat = offs.reshape(-1)
    de = emb_table.shape[1]
    emb_wide = emb_table.reshape(-1, 128)
    temb, embr = _sc_gather_call(h2, flat_ids, emb_wide, sf_flat, off_flat, de)

    x0 = jnp.concatenate([
        dense_features,
        temb.reshape(bsz, hist * d),
        embr.reshape(bsz, n_fields * emb_table.shape[1]),
    ], axis=1)

    dc = x0.shape[1]
    dc_pad = _ceil_to(dc, 128)
    x0p = jnp.pad(x0, ((0, 0), (0, dc_pad - dc)))
    cwp = jnp.pad(cross_w, ((0, 0), (0, dc_pad - dc)))
    cbp = jnp.pad(cross_b, ((0, 0), (0, dc_pad - dc)))
    w1p = jnp.pad(W_mlp1, ((0, dc_pad - dc), (0, 0)))
    wop = jnp.pad(W_out, ((0, 0), (0, 127)))
    bop = jnp.pad(b_out, (0, 127))

    out = _tc_head(x0p, cwp, cbp, w1p, b_mlp1, W_mlp2, b_mlp2, wop, bop)
    return out[:, :1]
